# baseline (speedup 1.0000x reference)
import jax
import jax.numpy as jnp
from jax import lax
from jax.experimental import pallas as pl
from jax.experimental.pallas import tpu as pltpu
from jax.experimental.pallas import tpu_sc as plsc


def _body(idx_hbm, tab_hbm, out_hbm, idx_v, row_v, sem):
    pltpu.sync_copy(idx_hbm.at[pl.ds(0, 128)], idx_v.at[0])
    vvec = idx_v[0, pl.ds(0, 16)]
    v = vvec[5]
    pltpu.async_copy(tab_hbm.at[3, pl.ds(v, 1), :], row_v, sem).wait()
    acc = row_v[0, pl.ds(0, 16)]
    out_v16 = acc + acc
    out_hbm_slice = out_hbm.at[pl.ds(0, 8)]
    pltpu.sync_copy(row_v, out_hbm_slice.at[pl.ds(0, 1), pl.ds(0, 32)])
    del out_v16


@jax.jit
def kernel(x, tables):
    offs = (jnp.arange(26, dtype=jnp.int32) * 100000)[None, :]
    idx = (x.astype(jnp.int32) + offs).reshape(-1)
    tab2 = tables
    run = pl.kernel(
        _body,
        mesh=plsc.VectorSubcoreMesh(core_axis_name="c", subcore_axis_name="s"),
        compiler_params=pltpu.CompilerParams(use_tc_tiling_on_sc=True),
        out_type=jax.ShapeDtypeStruct((16384, 32), jnp.float32),
        scratch_types=[
            pltpu.VMEM((1, 128), jnp.int32),
            pltpu.VMEM((1, 32), jnp.float32),
            pltpu.SemaphoreType.DMA,
        ],
    )
    return run(idx, tab2)


# speedup vs baseline: 1.4733x; 1.4733x over previous
import jax
import jax.numpy as jnp
from jax import lax
from jax.experimental import pallas as pl
from jax.experimental.pallas import tpu as pltpu
from jax.experimental.pallas import tpu_sc as plsc

_N_FIELDS = 26
_FIELDS_PAD = 32
_VOCAB = 100000
_EMB = 32
_BATCH = 16384
_LANES = 16

_NC = 2
_NS = 16
_NW = _NC * _NS
_ROWS_PER_W = _BATCH // _NW
_CHUNK = 32
_N_CHUNKS = _ROWS_PER_W // _CHUNK
_IDX_ROWS = _CHUNK * _FIELDS_PAD // 128
_CROWS = _CHUNK * _N_FIELDS


def _body(idx_hbm, tab_hbm, out_hbm, idx_v, rows_v, outb_v, sem):
    wid = lax.axis_index("s") * _NC + lax.axis_index("c")

    def step(t, carry):
        pltpu.sync_copy(
            idx_hbm.at[pl.ds(wid * (_ROWS_PER_W * _FIELDS_PAD // 128)
                             + t * _IDX_ROWS, _IDX_ROWS)], idx_v)
        for b in range(_CHUNK):
            base = b * _FIELDS_PAD
            v0 = idx_v[base // 128, pl.ds(base % 128, _LANES)]
            v1 = idx_v[base // 128, pl.ds(base % 128 + _LANES, _LANES)]
            for f in range(_N_FIELDS):
                v = v0[f] if f < _LANES else v1[f - _LANES]
                pltpu.async_copy(
                    tab_hbm.at[pl.ds(v, 1), :],
                    rows_v.at[pl.ds(b * _N_FIELDS + f, 1)], sem)
        pltpu.make_async_copy(tab_hbm.at[pl.ds(0, _CROWS)], rows_v, sem).wait()
        for b in range(_CHUNK):
            for h in range(_EMB // _LANES):
                sl = pl.ds(h * _LANES, _LANES)
                acc = rows_v[b * _N_FIELDS, sl]
                for f in range(1, _N_FIELDS):
                    acc = acc + rows_v[b * _N_FIELDS + f, sl]
                outb_v[b, sl] = acc
        pltpu.sync_copy(
            outb_v, out_hbm.at[pl.ds(wid * _ROWS_PER_W + t * _CHUNK, _CHUNK)])
        return carry

    lax.fori_loop(0, _N_CHUNKS, step, 0)


@jax.jit
def kernel(x, tables):
    offs = (jnp.arange(_N_FIELDS, dtype=jnp.int32) * _VOCAB)[None, :]
    flat = x.astype(jnp.int32) + offs
    idxp = jnp.concatenate(
        [flat, jnp.zeros((_BATCH, _FIELDS_PAD - _N_FIELDS), jnp.int32)],
        axis=1).reshape(_BATCH * _FIELDS_PAD // 128, 128)
    tab2 = tables.reshape(_N_FIELDS * _VOCAB, _EMB)
    run = pl.kernel(
        _body,
        mesh=plsc.VectorSubcoreMesh(core_axis_name="c", subcore_axis_name="s"),
        compiler_params=pltpu.CompilerParams(use_tc_tiling_on_sc=True),
        out_type=jax.ShapeDtypeStruct((_BATCH, _EMB), jnp.float32),
        scratch_types=[
            pltpu.VMEM((_IDX_ROWS, 128), jnp.int32),
            pltpu.VMEM((_CROWS, _EMB), jnp.float32),
            pltpu.VMEM((_CHUNK, _EMB), jnp.float32),
            pltpu.SemaphoreType.DMA,
        ],
    )
    return run(idxp, tab2)
